# Initial kernel scaffold; baseline (speedup 1.0000x reference)
import jax
import jax.numpy as jnp
from jax.experimental import pallas as pl


def kernel(user_ids, item_ids, emb_user_mf, emb_item_mf, emb_user_mlp, emb_item_mlp, W0, b0, W1, b1, Wl, bl):
    raise NotImplementedError("write your pallas kernel here")


# speedup vs baseline: 4.7815x; 4.7815x over previous
import functools

import jax
import jax.numpy as jnp
from jax import lax
from jax.experimental import pallas as pl
from jax.experimental.pallas import tpu as pltpu
from jax.experimental.pallas import tpu_sc as plsc

B = 16384
L = 20
F = 32
D1 = 32
D2 = 16

NC = 2
NS = 16
NW = NC * NS

BL = B * L
RI = BL // NW
RU = B // NW
CH = 128
GRP = 16
ROWS_G = CH * GRP
NGI = RI // ROWS_G
NCH_I = RI // CH
NCH_U = RU // CH


def _sc_gather_body(itab_mf, itab_mlp, utab_mf, utab_mlp, iidx_hbm, uidx_hbm,
                    o_imf, o_imlp, o_umf, o_umlp,
                    iidx_v, uidx_v, rbuf, ubuf, sem):
  wid = lax.axis_index("s") * NC + lax.axis_index("c")

  pltpu.sync_copy(iidx_hbm.at[wid], iidx_v)
  pltpu.sync_copy(uidx_hbm.at[wid], uidx_v)

  ibase = wid * RI
  ubase = wid * RU

  def item_table(tab, out):
    def group(g, carry):
      descs = []
      for i in range(GRP):
        d = pltpu.async_copy(
            tab.at[iidx_v.at[g * GRP + i]],
            rbuf.at[pl.ds(i * CH, CH)], sem)
        descs.append(d)
      for d in descs:
        d.wait()
      pltpu.sync_copy(rbuf, out.at[pl.ds(ibase + g * ROWS_G, ROWS_G)])
      return carry
    lax.fori_loop(0, NGI, group, 0)

  def user_table(tab, out):
    descs = []
    for i in range(NCH_U):
      d = pltpu.async_copy(
          tab.at[uidx_v.at[i]],
          ubuf.at[pl.ds(i * CH, CH)], sem)
      descs.append(d)
    for d in descs:
      d.wait()
    pltpu.sync_copy(ubuf, out.at[pl.ds(ubase, RU)])

  item_table(itab_mf, o_imf)
  item_table(itab_mlp, o_imlp)
  user_table(utab_mf, o_umf)
  user_table(utab_mlp, o_umlp)


def _sc_gather(itab_mf, itab_mlp, utab_mf, utab_mlp, iidx, uidx):
  mesh = plsc.VectorSubcoreMesh(core_axis_name="c", subcore_axis_name="s")
  return pl.kernel(
      _sc_gather_body,
      out_type=[
          jax.ShapeDtypeStruct((BL, F), jnp.float32),
          jax.ShapeDtypeStruct((BL, F), jnp.float32),
          jax.ShapeDtypeStruct((B, F), jnp.float32),
          jax.ShapeDtypeStruct((B, F), jnp.float32),
      ],
      mesh=mesh,
      scratch_types=[
          pltpu.VMEM((NCH_I, CH), jnp.int32),
          pltpu.VMEM((NCH_U, CH), jnp.int32),
          pltpu.VMEM((ROWS_G, F), jnp.float32),
          pltpu.VMEM((RU, F), jnp.float32),
          pltpu.SemaphoreType.DMA,
      ],
      compiler_params=pltpu.CompilerParams(use_tc_tiling_on_sc=False),
  )(itab_mf, itab_mlp, utab_mf, utab_mlp, iidx, uidx)


BBLK = 512


def _tc_body(gim, gif, gum, guf, w0, b0, w1, b1, wlmf, wlmlp, bl, out):
  w0u = w0[0:F, :]
  w0i = w0[F:2 * F, :]
  pu0 = jnp.dot(gum[...], w0u, preferred_element_type=jnp.float32) + b0[...]
  v = guf[...] * wlmf[...]
  cols = []
  for l in range(L):
    xm = gim[l]
    xf = gif[l]
    h0 = jnp.maximum(
        jnp.dot(xm, w0i, preferred_element_type=jnp.float32) + pu0, 0.0)
    h1 = jnp.maximum(
        jnp.dot(h0, w1[...], preferred_element_type=jnp.float32) + b1[...], 0.0)
    mo = jnp.dot(h1, wlmlp[...], preferred_element_type=jnp.float32)
    fo = jnp.sum(xf * v, axis=1, keepdims=True)
    cols.append(mo + fo)
  out[...] = jnp.concatenate(cols, axis=1) + bl[...]


def _tc_tower(gim, gif, gum, guf, W0, b0, W1, b1, wlmf, wlmlp, bl):
  grid = (B // BBLK,)
  return pl.pallas_call(
      _tc_body,
      grid=grid,
      in_specs=[
          pl.BlockSpec((L, BBLK, F), lambda i: (0, i, 0)),
          pl.BlockSpec((L, BBLK, F), lambda i: (0, i, 0)),
          pl.BlockSpec((BBLK, F), lambda i: (i, 0)),
          pl.BlockSpec((BBLK, F), lambda i: (i, 0)),
          pl.BlockSpec((2 * F, D1), lambda i: (0, 0)),
          pl.BlockSpec((1, D1), lambda i: (0, 0)),
          pl.BlockSpec((D1, D2), lambda i: (0, 0)),
          pl.BlockSpec((1, D2), lambda i: (0, 0)),
          pl.BlockSpec((1, F), lambda i: (0, 0)),
          pl.BlockSpec((D2, 1), lambda i: (0, 0)),
          pl.BlockSpec((1, 1), lambda i: (0, 0)),
      ],
      out_specs=pl.BlockSpec((BBLK, L), lambda i: (i, 0)),
      out_shape=jax.ShapeDtypeStruct((B, L), jnp.float32),
  )(gim, gif, gum, guf, W0, b0, W1, b1, wlmf, wlmlp, bl)


@jax.jit
def kernel(user_ids, item_ids, emb_user_mf, emb_item_mf, emb_user_mlp,
           emb_item_mlp, W0, b0, W1, b1, Wl, bl):
  iidx = jnp.transpose(item_ids.astype(jnp.int32)).reshape(NW, NCH_I, CH)
  uidx = user_ids.astype(jnp.int32).reshape(NW, NCH_U, CH)

  g_imf, g_imlp, g_umf, g_umlp = _sc_gather(
      emb_item_mf, emb_item_mlp, emb_user_mf, emb_user_mlp, iidx, uidx)

  gim = g_imlp.reshape(L, B, F)
  gif = g_imf.reshape(L, B, F)

  out = _tc_tower(
      gim, gif, g_umlp, g_umf,
      W0, b0.reshape(1, D1), W1, b1.reshape(1, D2),
      Wl[:F].reshape(1, F), Wl[F:], bl.reshape(1, 1))
  return out


# speedup vs baseline: 5.4742x; 1.1449x over previous
import jax
import jax.numpy as jnp
from jax import lax
from jax.experimental import pallas as pl
from jax.experimental.pallas import tpu as pltpu
from jax.experimental.pallas import tpu_sc as plsc

B = 16384
L = 20
F = 32
D1 = 32
D2 = 16

NC = 2
NS = 16
NW = NC * NS

BL = B * L
RI = BL // NW
RU = B // NW
CH = 128
GRP = 16
ROWS_G = CH * GRP
NGI = RI // ROWS_G
NCH_I = RI // CH
NCH_U = RU // CH


def _sc_gather_body(itab_mf, itab_mlp, utab_mf, utab_mlp, iidx_hbm, uidx_hbm,
                    o_imf, o_imlp, o_umf, o_umlp,
                    iidx_v, uidx_v, rbuf, ubuf, sem):
  wid = lax.axis_index("s") * NC + lax.axis_index("c")

  pltpu.sync_copy(iidx_hbm.at[wid], iidx_v)
  pltpu.sync_copy(uidx_hbm.at[wid], uidx_v)

  ibase = wid * RI
  ubase = wid * RU

  def item_table(tab, out):
    def group(g, carry):
      descs = []
      for i in range(GRP):
        d = pltpu.async_copy(
            tab.at[iidx_v.at[g * GRP + i]],
            rbuf.at[pl.ds(i * CH, CH)], sem)
        descs.append(d)
      for d in descs:
        d.wait()
      pltpu.sync_copy(rbuf, out.at[pl.ds(ibase + g * ROWS_G, ROWS_G)])
      return carry
    lax.fori_loop(0, NGI, group, 0)

  def user_table(tab, out):
    descs = []
    for i in range(NCH_U):
      d = pltpu.async_copy(
          tab.at[uidx_v.at[i]],
          ubuf.at[pl.ds(i * CH, CH)], sem)
      descs.append(d)
    for d in descs:
      d.wait()
    pltpu.sync_copy(ubuf, out.at[pl.ds(ubase, RU)])

  item_table(itab_mf, o_imf)
  item_table(itab_mlp, o_imlp)
  user_table(utab_mf, o_umf)
  user_table(utab_mlp, o_umlp)


def _sc_gather(itab_mf, itab_mlp, utab_mf, utab_mlp, iidx, uidx):
  mesh = plsc.VectorSubcoreMesh(core_axis_name="c", subcore_axis_name="s")
  return pl.kernel(
      _sc_gather_body,
      out_type=[
          jax.ShapeDtypeStruct((BL, F), jnp.float32),
          jax.ShapeDtypeStruct((BL, F), jnp.float32),
          jax.ShapeDtypeStruct((B, F), jnp.float32),
          jax.ShapeDtypeStruct((B, F), jnp.float32),
      ],
      mesh=mesh,
      scratch_types=[
          pltpu.VMEM((NCH_I, CH), jnp.int32),
          pltpu.VMEM((NCH_U, CH), jnp.int32),
          pltpu.VMEM((ROWS_G, F), jnp.float32),
          pltpu.VMEM((RU, F), jnp.float32),
          pltpu.SemaphoreType.DMA,
      ],
      compiler_params=pltpu.CompilerParams(use_tc_tiling_on_sc=False),
  )(itab_mf, itab_mlp, utab_mf, utab_mlp, iidx, uidx)


BBLK = 512
PK = 4
RPB = L // PK


def _tc_body(x4, xf4, gum, guf, w0ut, b0t, w0ib, w1b, b1t, wlmf, wsel1, wsel2,
             tile4, blr, out):
  pu_t = jnp.dot(gum[...], w0ut[...], preferred_element_type=jnp.float32) \
      + b0t[...]
  v_t = jnp.dot(guf[...] * wlmf[...], tile4[...],
                preferred_element_type=jnp.float32)
  pu_r = jnp.broadcast_to(pu_t[None], (RPB, BBLK, PK * F)).reshape(
      RPB * BBLK, PK * F)
  v_r = jnp.broadcast_to(v_t[None], (RPB, BBLK, PK * F)).reshape(
      RPB * BBLK, PK * F)
  xx = x4[...].reshape(RPB * BBLK, PK * F)
  xf = xf4[...].reshape(RPB * BBLK, PK * F)
  h0 = jnp.maximum(
      jnp.dot(xx, w0ib[...], preferred_element_type=jnp.float32) + pu_r,
      0.0)
  h1 = jnp.maximum(
      jnp.dot(h0, w1b[...], preferred_element_type=jnp.float32) + b1t[...],
      0.0)
  res = (jnp.dot(h1, wsel1[...], preferred_element_type=jnp.float32)
         + jnp.dot(xf * v_r, wsel2[...],
                   preferred_element_type=jnp.float32) + blr[...])
  out[...] = res.reshape(RPB, BBLK, PK)


def _tc_tower(g_imlp, g_imf, g_umlp, g_umf, W0, b0, W1, b1, Wl, bl):
  x4 = g_imlp.reshape(RPB, B, PK * F)
  xf4 = g_imf.reshape(RPB, B, PK * F)
  eyef = jnp.eye(F, dtype=jnp.float32)
  eyep = jnp.eye(PK, dtype=jnp.float32)
  tile4 = jnp.tile(eyef, (1, PK))
  w0ut = jnp.tile(W0[:F], (1, PK))
  b0t = jnp.tile(b0, PK).reshape(1, PK * D1)
  w0ib = jnp.kron(eyep, W0[F:])
  w1b = jnp.kron(eyep, W1)
  b1t = jnp.tile(b1, PK).reshape(1, PK * D2)
  wlmf = Wl[:F].reshape(1, F)
  wsel1 = jnp.kron(eyep, Wl[F:])
  wsel2 = jnp.kron(eyep, jnp.ones((F, 1), jnp.float32))
  blr = bl.reshape(1, 1)

  out = pl.pallas_call(
      _tc_body,
      grid=(B // BBLK,),
      in_specs=[
          pl.BlockSpec((RPB, BBLK, PK * F), lambda i: (0, i, 0)),
          pl.BlockSpec((RPB, BBLK, PK * F), lambda i: (0, i, 0)),
          pl.BlockSpec((BBLK, F), lambda i: (i, 0)),
          pl.BlockSpec((BBLK, F), lambda i: (i, 0)),
          pl.BlockSpec((F, PK * F), lambda i: (0, 0)),
          pl.BlockSpec((1, PK * D1), lambda i: (0, 0)),
          pl.BlockSpec((PK * F, PK * D1), lambda i: (0, 0)),
          pl.BlockSpec((PK * D1, PK * D2), lambda i: (0, 0)),
          pl.BlockSpec((1, PK * D2), lambda i: (0, 0)),
          pl.BlockSpec((1, F), lambda i: (0, 0)),
          pl.BlockSpec((PK * D2, PK), lambda i: (0, 0)),
          pl.BlockSpec((PK * F, PK), lambda i: (0, 0)),
          pl.BlockSpec((F, PK * F), lambda i: (0, 0)),
          pl.BlockSpec((1, 1), lambda i: (0, 0)),
      ],
      out_specs=pl.BlockSpec((RPB, BBLK, PK), lambda i: (0, i, 0)),
      out_shape=jax.ShapeDtypeStruct((RPB, B, PK), jnp.float32),
  )(x4, xf4, g_umlp, g_umf, w0ut, b0t, w0ib, w1b, b1t, wlmf, wsel1, wsel2,
    tile4, blr)
  return out.transpose(1, 0, 2).reshape(B, L)


@jax.jit
def kernel(user_ids, item_ids, emb_user_mf, emb_item_mf, emb_user_mlp,
           emb_item_mlp, W0, b0, W1, b1, Wl, bl):
  iidx = item_ids.astype(jnp.int32).reshape(B, RPB, PK).transpose(
      1, 0, 2).reshape(NW, NCH_I, CH)
  uidx = user_ids.astype(jnp.int32).reshape(NW, NCH_U, CH)

  g_imf, g_imlp, g_umf, g_umlp = _sc_gather(
      emb_item_mf, emb_item_mlp, emb_user_mf, emb_user_mlp, iidx, uidx)

  return _tc_tower(g_imlp, g_imf, g_umlp, g_umf, W0, b0, W1, b1, Wl, bl)
